# baseline (speedup 1.0000x reference)
import functools

import jax
import jax.numpy as jnp
from jax.experimental import pallas as pl
from jax.experimental.pallas import tpu as pltpu

D_MODEL = 1024
FFN = 3584
N_EXP = 8
T = 2048
A = 2 * T

TT = 128
NT = A // TT + N_EXP
P = NT * TT
F_BLK = 256
N_F = FFN // F_BLK


def _router_kernel(x_ref, wg_ref, sel_ref, wsel_ref):
    x = x_ref[...]
    wg = wg_ref[...]
    logits = jax.lax.dot_general(
        x, wg, (((1,), (1,)), ((), ())),
        precision=jax.lax.Precision.DEFAULT,
        preferred_element_type=jnp.float32,
    )
    neg = jnp.float32(-jnp.inf)
    m1 = jnp.max(logits, axis=1, keepdims=True)
    lane = jax.lax.broadcasted_iota(jnp.int32, logits.shape, 1)
    e1 = jnp.min(jnp.where(logits == m1, lane, N_EXP), axis=1, keepdims=True)
    masked = jnp.where(lane == e1, neg, logits)
    m2 = jnp.max(masked, axis=1, keepdims=True)
    e2 = jnp.min(jnp.where(masked == m2, lane, N_EXP), axis=1, keepdims=True)
    w_hi = 1.0 / (1.0 + jnp.exp(m2 - m1))
    w_lo = 1.0 - w_hi
    sel_ref[...] = jnp.concatenate([e1, e2], axis=1)
    wsel_ref[...] = jnp.concatenate([w_hi, w_lo], axis=1)


def _group_metadata(sel, wsel):
    ef = sel.reshape(-1).astype(jnp.int32)
    wf = wsel.reshape(-1)
    tokf = jnp.arange(A, dtype=jnp.int32) // 2
    order = jnp.argsort(ef)
    ef_s = ef[order]
    n_e = jnp.bincount(ef, length=N_EXP).astype(jnp.int32)
    p_e = ((n_e + TT - 1) // TT) * TT
    S = jnp.concatenate([jnp.zeros(1, jnp.int32), jnp.cumsum(p_e)])
    o = jnp.concatenate([jnp.zeros(1, jnp.int32), jnp.cumsum(n_e)])
    k = jnp.arange(A, dtype=jnp.int32)
    pos = S[ef_s] + (k - o[ef_s])
    sorted_tok = jnp.full((P,), T, jnp.int32).at[pos].set(tokf[order])
    w_sorted = jnp.zeros((P,), jnp.float32).at[pos].set(wf[order])
    starts = jnp.arange(NT, dtype=jnp.int32) * TT
    te = jnp.clip(jnp.searchsorted(S[1:], starts, side="right"), 0,
                  N_EXP - 1).astype(jnp.int32)
    tv = (starts < S[N_EXP]).astype(jnp.int32)
    return (sorted_tok.reshape(NT, 1, TT), w_sorted.reshape(NT, 1, TT), te, tv)


def _onehot_t(tok_row):
    iota0 = jax.lax.broadcasted_iota(jnp.int32, (T, TT), 0)
    return (iota0 == tok_row).astype(jnp.bfloat16)


def _moe_group_kernel(te_ref, tv_ref, xb_ref, tok_ref, w_ref,
                      w1_ref, w3_ref, w2_ref, out_ref, xs_ref, acc_ref):
    f = pl.program_id(0)
    i = pl.program_id(1)

    @pl.when(jnp.logical_and(f == 0, i == 0))
    def _init():
        out_ref[...] = jnp.zeros_like(out_ref)

    @pl.when(tv_ref[i] > 0)
    def _body():
        rows = pl.ds(i * TT, TT)

        @pl.when(f == 0)
        def _gather():
            oh = _onehot_t(tok_ref[0])
            xg = jax.lax.dot_general(oh, xb_ref[...], (((0,), (0,)), ((), ())),
                                     preferred_element_type=jnp.float32)
            xs_ref[rows, :] = xg.astype(jnp.bfloat16)

        xg = xs_ref[rows, :]
        w1 = w1_ref[0].astype(jnp.bfloat16)
        w3 = w3_ref[0].astype(jnp.bfloat16)
        w2 = w2_ref[0].astype(jnp.bfloat16)
        h1 = jax.lax.dot_general(xg, w1, (((1,), (1,)), ((), ())),
                                 preferred_element_type=jnp.float32)
        h3 = jax.lax.dot_general(xg, w3, (((1,), (1,)), ((), ())),
                                 preferred_element_type=jnp.float32)
        h = ((h1 * jax.nn.sigmoid(h1)) * h3).astype(jnp.bfloat16)
        y = jax.lax.dot_general(h, w2, (((1,), (1,)), ((), ())),
                                preferred_element_type=jnp.float32)

        @pl.when(f == 0)
        def _store():
            acc_ref[rows, :] = y

        @pl.when(f > 0)
        def _accum():
            acc_ref[rows, :] += y

        @pl.when(f == N_F - 1)
        def _scatter():
            ohw = _onehot_t(tok_ref[0]) * w_ref[0].astype(jnp.bfloat16)
            accb = acc_ref[rows, :].astype(jnp.bfloat16)
            for c in range(4):
                seg = pl.ds(c * (T // 4), T // 4)
                out_ref[seg, :] += jax.lax.dot_general(
                    ohw[c * (T // 4):(c + 1) * (T // 4), :], accb,
                    (((1,), (0,)), ((), ())),
                    preferred_element_type=jnp.float32)


@functools.partial(jax.jit, static_argnames=("interpret",))
def _run(x, Wg, W1, W3, W2, interpret=False):
    sel, wsel = pl.pallas_call(
        _router_kernel,
        out_shape=(jax.ShapeDtypeStruct((T, 2), jnp.int32),
                   jax.ShapeDtypeStruct((T, 2), jnp.float32)),
        interpret=interpret,
    )(x, Wg)

    sorted_tok, w_sorted, te, tv = _group_metadata(sel, wsel)
    xb = x.astype(jnp.bfloat16)

    out = pl.pallas_call(
        _moe_group_kernel,
        grid_spec=pltpu.PrefetchScalarGridSpec(
            num_scalar_prefetch=2,
            grid=(N_F, NT),
            in_specs=[
                pl.BlockSpec((T, D_MODEL), lambda f, i, te, tv: (0, 0)),
                pl.BlockSpec((1, 1, TT), lambda f, i, te, tv: (i, 0, 0)),
                pl.BlockSpec((1, 1, TT), lambda f, i, te, tv: (i, 0, 0)),
                pl.BlockSpec((1, F_BLK, D_MODEL),
                             lambda f, i, te, tv: (te[i], f, 0)),
                pl.BlockSpec((1, F_BLK, D_MODEL),
                             lambda f, i, te, tv: (te[i], f, 0)),
                pl.BlockSpec((1, D_MODEL, F_BLK),
                             lambda f, i, te, tv: (te[i], 0, f)),
            ],
            out_specs=pl.BlockSpec((T, D_MODEL), lambda f, i, te, tv: (0, 0)),
            scratch_shapes=[
                pltpu.VMEM((P, D_MODEL), jnp.bfloat16),
                pltpu.VMEM((P, D_MODEL), jnp.float32),
            ],
        ),
        out_shape=jax.ShapeDtypeStruct((T, D_MODEL), jnp.float32),
        interpret=interpret,
    )(te, tv, xb, sorted_tok, w_sorted, W1, W3, W2)
    return out


def kernel(hidden_states, Wg, W1, W3, W2):
    b, s, d = hidden_states.shape
    x = hidden_states.reshape(-1, d)
    out = _run(x, Wg, W1, W3, W2)
    return out.reshape(b, s, d)


# speedup vs baseline: 1.6158x; 1.6158x over previous
import functools

import jax
import jax.numpy as jnp
from jax.experimental import pallas as pl
from jax.experimental.pallas import tpu as pltpu

D_MODEL = 1024
FFN = 3584
N_EXP = 8
T = 2048
A = 2 * T

TT = 256
NT = A // TT + N_EXP
P = NT * TT
F_BLK = 512
N_F = FFN // F_BLK
TC = 1024


def _router_kernel(x_ref, wg_ref, sel_ref, wsel_ref):
    x = x_ref[...]
    wg = wg_ref[...]
    logits = jax.lax.dot_general(
        x, wg, (((1,), (1,)), ((), ())),
        precision=jax.lax.Precision.DEFAULT,
        preferred_element_type=jnp.float32,
    )
    neg = jnp.float32(-jnp.inf)
    m1 = jnp.max(logits, axis=1, keepdims=True)
    lane = jax.lax.broadcasted_iota(jnp.int32, logits.shape, 1)
    e1 = jnp.min(jnp.where(logits == m1, lane, N_EXP), axis=1, keepdims=True)
    masked = jnp.where(lane == e1, neg, logits)
    m2 = jnp.max(masked, axis=1, keepdims=True)
    e2 = jnp.min(jnp.where(masked == m2, lane, N_EXP), axis=1, keepdims=True)
    w_hi = 1.0 / (1.0 + jnp.exp(m2 - m1))
    w_lo = 1.0 - w_hi
    sel_ref[...] = jnp.concatenate([e1, e2], axis=1)
    wsel_ref[...] = jnp.concatenate([w_hi, w_lo], axis=1)


def _group_metadata(sel, wsel):
    ef = sel.reshape(-1).astype(jnp.int32)
    wf = wsel.reshape(-1)
    tokf = jnp.arange(A, dtype=jnp.int32) // 2
    order = jnp.argsort(ef)
    ef_s = ef[order]
    n_e = jnp.bincount(ef, length=N_EXP).astype(jnp.int32)
    p_e = ((n_e + TT - 1) // TT) * TT
    S = jnp.concatenate([jnp.zeros(1, jnp.int32), jnp.cumsum(p_e)])
    o = jnp.concatenate([jnp.zeros(1, jnp.int32), jnp.cumsum(n_e)])
    k = jnp.arange(A, dtype=jnp.int32)
    pos = S[ef_s] + (k - o[ef_s])
    sorted_tok = jnp.full((P,), T, jnp.int32).at[pos].set(tokf[order])
    w_sorted = jnp.zeros((P,), jnp.float32).at[pos].set(wf[order])
    starts = jnp.arange(NT, dtype=jnp.int32) * TT
    te = jnp.clip(jnp.searchsorted(S[1:], starts, side="right"), 0,
                  N_EXP - 1).astype(jnp.int32)
    tv = (starts < S[N_EXP]).astype(jnp.int32)
    return (sorted_tok.reshape(NT, 1, TT), sorted_tok.reshape(1, P),
            w_sorted.reshape(1, P), te, tv)


def _gather_kernel(tok_ref, xb_ref, xs_ref):
    iota0 = jax.lax.broadcasted_iota(jnp.int32, (T, TT), 0)
    oh = (iota0 == tok_ref[0]).astype(jnp.bfloat16)
    xg = jax.lax.dot_general(oh, xb_ref[...], (((0,), (0,)), ((), ())),
                             preferred_element_type=jnp.float32)
    xs_ref[...] = xg.astype(jnp.bfloat16)


def _ffn_kernel(te_ref, tv_ref, xs_ref, w1_ref, w3_ref, w2_ref,
                ys_ref, acc_ref):
    f = pl.program_id(0)
    i = pl.program_id(1)
    valid = tv_ref[i] > 0
    rows = pl.ds(i * TT, TT)

    @pl.when(valid)
    def _body():
        xg = xs_ref[rows, :]
        w1 = w1_ref[0].astype(jnp.bfloat16)
        w3 = w3_ref[0].astype(jnp.bfloat16)
        w2 = w2_ref[0].astype(jnp.bfloat16)
        h1 = jax.lax.dot_general(xg, w1, (((1,), (1,)), ((), ())),
                                 preferred_element_type=jnp.float32)
        h3 = jax.lax.dot_general(xg, w3, (((1,), (1,)), ((), ())),
                                 preferred_element_type=jnp.float32)
        h = ((h1 * jax.nn.sigmoid(h1)) * h3).astype(jnp.bfloat16)
        y = jax.lax.dot_general(h, w2, (((1,), (1,)), ((), ())),
                                preferred_element_type=jnp.float32)

        @pl.when(f == 0)
        def _store():
            acc_ref[rows, :] = y

        @pl.when(f > 0)
        def _accum():
            acc_ref[rows, :] += y

    @pl.when(jnp.logical_and(jnp.logical_not(valid), f == 0))
    def _zero():
        acc_ref[rows, :] = jnp.zeros((TT, D_MODEL), jnp.float32)

    ys_ref[...] = acc_ref[rows, :].astype(jnp.bfloat16)


def _combine_kernel(tok_ref, w_ref, ys_ref, out_ref, ohw_ref):
    t = pl.program_id(0)
    base = t * TC
    n_chunks = P // TC
    for c in range(n_chunks):
        cols = pl.ds(c * TC, TC)
        iota0 = jax.lax.broadcasted_iota(jnp.int32, (TC, TC), 0) + base
        tok_c = tok_ref[0, cols].reshape(1, TC)
        w_c = w_ref[0, cols].reshape(1, TC)
        ohw_ref[:, cols] = jnp.where(iota0 == tok_c, w_c,
                                     0.0).astype(jnp.bfloat16)
    out_ref[...] = jax.lax.dot_general(
        ohw_ref[...], ys_ref[...], (((1,), (0,)), ((), ())),
        preferred_element_type=jnp.float32)


@functools.partial(jax.jit, static_argnames=("interpret",))
def _run(x, Wg, W1, W3, W2, interpret=False):
    sel, wsel = pl.pallas_call(
        _router_kernel,
        out_shape=(jax.ShapeDtypeStruct((T, 2), jnp.int32),
                   jax.ShapeDtypeStruct((T, 2), jnp.float32)),
        interpret=interpret,
    )(x, Wg)

    tok3d, tok2d, w2d, te, tv = _group_metadata(sel, wsel)
    xb = x.astype(jnp.bfloat16)

    xs = pl.pallas_call(
        _gather_kernel,
        grid=(NT,),
        in_specs=[
            pl.BlockSpec((1, 1, TT), lambda i: (i, 0, 0)),
            pl.BlockSpec((T, D_MODEL), lambda i: (0, 0)),
        ],
        out_specs=pl.BlockSpec((TT, D_MODEL), lambda i: (i, 0)),
        out_shape=jax.ShapeDtypeStruct((P, D_MODEL), jnp.bfloat16),
        interpret=interpret,
    )(tok3d, xb)

    ys = pl.pallas_call(
        _ffn_kernel,
        grid_spec=pltpu.PrefetchScalarGridSpec(
            num_scalar_prefetch=2,
            grid=(N_F, NT),
            in_specs=[
                pl.BlockSpec((P, D_MODEL), lambda f, i, te, tv: (0, 0)),
                pl.BlockSpec((1, F_BLK, D_MODEL),
                             lambda f, i, te, tv: (te[i], f, 0)),
                pl.BlockSpec((1, F_BLK, D_MODEL),
                             lambda f, i, te, tv: (te[i], f, 0)),
                pl.BlockSpec((1, D_MODEL, F_BLK),
                             lambda f, i, te, tv: (te[i], 0, f)),
            ],
            out_specs=pl.BlockSpec((TT, D_MODEL), lambda f, i, te, tv: (i, 0)),
            scratch_shapes=[pltpu.VMEM((P, D_MODEL), jnp.float32)],
        ),
        out_shape=jax.ShapeDtypeStruct((P, D_MODEL), jnp.bfloat16),
        interpret=interpret,
    )(te, tv, xs, W1, W3, W2)

    out = pl.pallas_call(
        _combine_kernel,
        grid=(T // TC,),
        in_specs=[
            pl.BlockSpec((1, P), lambda t: (0, 0)),
            pl.BlockSpec((1, P), lambda t: (0, 0)),
            pl.BlockSpec((P, D_MODEL), lambda t: (0, 0)),
        ],
        out_specs=pl.BlockSpec((TC, D_MODEL), lambda t: (t, 0)),
        out_shape=jax.ShapeDtypeStruct((T, D_MODEL), jnp.float32),
        scratch_shapes=[pltpu.VMEM((TC, P), jnp.bfloat16)],
        interpret=interpret,
    )(tok2d, w2d, ys)
    return out


def kernel(hidden_states, Wg, W1, W3, W2):
    b, s, d = hidden_states.shape
    x = hidden_states.reshape(-1, d)
    out = _run(x, Wg, W1, W3, W2)
    return out.reshape(b, s, d)


# speedup vs baseline: 1.8638x; 1.1535x over previous
import functools

import jax
import jax.numpy as jnp
from jax.experimental import pallas as pl
from jax.experimental.pallas import tpu as pltpu

D_MODEL = 1024
FFN = 3584
N_EXP = 8
T = 2048
A = 2 * T

TT = 512
NT = A // TT + N_EXP
P = NT * TT
F_BLK = 512
N_F = FFN // F_BLK
TC_CHUNK = 1024


def _router_kernel(x_ref, wg_ref, sel_ref, wsel_ref):
    x = x_ref[...]
    wg = wg_ref[...]
    logits = jax.lax.dot_general(
        x, wg, (((1,), (1,)), ((), ())),
        precision=jax.lax.Precision.DEFAULT,
        preferred_element_type=jnp.float32,
    )
    neg = jnp.float32(-jnp.inf)
    m1 = jnp.max(logits, axis=1, keepdims=True)
    lane = jax.lax.broadcasted_iota(jnp.int32, logits.shape, 1)
    e1 = jnp.min(jnp.where(logits == m1, lane, N_EXP), axis=1, keepdims=True)
    masked = jnp.where(lane == e1, neg, logits)
    m2 = jnp.max(masked, axis=1, keepdims=True)
    e2 = jnp.min(jnp.where(masked == m2, lane, N_EXP), axis=1, keepdims=True)
    w_hi = 1.0 / (1.0 + jnp.exp(m2 - m1))
    w_lo = 1.0 - w_hi
    sel_ref[...] = jnp.concatenate([e1, e2], axis=1)
    wsel_ref[...] = jnp.concatenate([w_hi, w_lo], axis=1)


def _group_metadata(sel, wsel):
    ef = sel.reshape(-1).astype(jnp.int32)
    wf = wsel.reshape(-1)
    tokf = jnp.arange(A, dtype=jnp.int32) // 2
    order = jnp.argsort(ef)
    ef_s = ef[order]
    n_e = jnp.bincount(ef, length=N_EXP).astype(jnp.int32)
    p_e = ((n_e + TT - 1) // TT) * TT
    S = jnp.concatenate([jnp.zeros(1, jnp.int32), jnp.cumsum(p_e)])
    o = jnp.concatenate([jnp.zeros(1, jnp.int32), jnp.cumsum(n_e)])
    k = jnp.arange(A, dtype=jnp.int32)
    pos = S[ef_s] + (k - o[ef_s])
    sorted_tok = jnp.full((P,), T, jnp.int32).at[pos].set(tokf[order])
    w_sorted = jnp.zeros((P,), jnp.float32).at[pos].set(wf[order])
    starts = jnp.arange(NT, dtype=jnp.int32) * TT
    te = jnp.clip(jnp.searchsorted(S[1:], starts, side="right"), 0,
                  N_EXP - 1).astype(jnp.int32)
    tv = (starts < S[N_EXP]).astype(jnp.int32)
    return sorted_tok, w_sorted, te, tv


def _gather_kernel(tok_ref, xb_ref, xs_ref):
    iota0 = jax.lax.broadcasted_iota(jnp.int32, (T, TT), 0)
    oh = (iota0 == tok_ref[0]).astype(jnp.bfloat16)
    xg = jax.lax.dot_general(oh, xb_ref[...], (((0,), (0,)), ((), ())),
                             preferred_element_type=jnp.float32)
    xs_ref[...] = xg.astype(jnp.bfloat16)


def _ffn_kernel(te_ref, tv_ref, xs_ref, w1_ref, w3_ref, w2_ref,
                ys_ref, acc_ref, w1b_ref, w3b_ref, w2b_ref):
    f = pl.program_id(0)
    i = pl.program_id(1)
    valid = tv_ref[i] > 0
    rows = pl.ds(i * TT, TT)

    same_as_prev = jnp.logical_and(
        i > 0, te_ref[jnp.maximum(i - 1, 0)] == te_ref[i])

    @pl.when(jnp.logical_not(same_as_prev))
    def _cast():
        w1b_ref[...] = w1_ref[0].astype(jnp.bfloat16)
        w3b_ref[...] = w3_ref[0].astype(jnp.bfloat16)
        w2b_ref[...] = w2_ref[0].astype(jnp.bfloat16)

    @pl.when(valid)
    def _body():
        xg = xs_ref[rows, :]
        h1 = jax.lax.dot_general(xg, w1b_ref[...], (((1,), (1,)), ((), ())),
                                 preferred_element_type=jnp.float32)
        h3 = jax.lax.dot_general(xg, w3b_ref[...], (((1,), (1,)), ((), ())),
                                 preferred_element_type=jnp.float32)
        h = ((h1 * jax.nn.sigmoid(h1)) * h3).astype(jnp.bfloat16)
        y = jax.lax.dot_general(h, w2b_ref[...], (((1,), (1,)), ((), ())),
                                preferred_element_type=jnp.float32)

        @pl.when(f == 0)
        def _store():
            acc_ref[rows, :] = y.astype(jnp.bfloat16)

        @pl.when(f > 0)
        def _accum():
            acc_ref[rows, :] = (
                acc_ref[rows, :].astype(jnp.float32) + y).astype(jnp.bfloat16)

    @pl.when(jnp.logical_and(jnp.logical_not(valid), f == 0))
    def _zero():
        acc_ref[rows, :] = jnp.zeros((TT, D_MODEL), jnp.bfloat16)

    @pl.when(f == N_F - 1)
    def _emit():
        ys_ref[...] = acc_ref[rows, :]


def _combine_kernel(tok_ref, w_ref, ys_ref, out_ref, ohw_ref):
    t = pl.program_id(0)
    base = t * TC_CHUNK
    for c in range(P // TC_CHUNK):
        cols = pl.ds(c * TC_CHUNK, TC_CHUNK)
        iota0 = jax.lax.broadcasted_iota(
            jnp.int32, (TC_CHUNK, TC_CHUNK), 0) + base
        tok_c = tok_ref[0, cols].reshape(1, TC_CHUNK)
        w_c = w_ref[0, cols].reshape(1, TC_CHUNK)
        ohw_ref[:, cols] = jnp.where(iota0 == tok_c, w_c,
                                     0.0).astype(jnp.bfloat16)
    out_ref[...] = jax.lax.dot_general(
        ohw_ref[...], ys_ref[...], (((1,), (0,)), ((), ())),
        preferred_element_type=jnp.float32)


@functools.partial(jax.jit, static_argnames=("interpret",))
def _run(x, Wg, W1, W3, W2, interpret=False):
    sel, wsel = pl.pallas_call(
        _router_kernel,
        out_shape=(jax.ShapeDtypeStruct((T, 2), jnp.int32),
                   jax.ShapeDtypeStruct((T, 2), jnp.float32)),
        interpret=interpret,
    )(x, Wg)

    sorted_tok, w_sorted, te, tv = _group_metadata(sel, wsel)

    tok3d = sorted_tok.reshape(NT, 1, TT)
    tok2d = sorted_tok.reshape(1, P)
    w2d = w_sorted.reshape(1, P)
    xb = x.astype(jnp.bfloat16)

    xs = pl.pallas_call(
        _gather_kernel,
        grid=(NT,),
        in_specs=[
            pl.BlockSpec((1, 1, TT), lambda i: (i, 0, 0)),
            pl.BlockSpec((T, D_MODEL), lambda i: (0, 0)),
        ],
        out_specs=pl.BlockSpec((TT, D_MODEL), lambda i: (i, 0)),
        out_shape=jax.ShapeDtypeStruct((P, D_MODEL), jnp.bfloat16),
        interpret=interpret,
    )(tok3d, xb)

    ys_padded = pl.pallas_call(
        _ffn_kernel,
        grid_spec=pltpu.PrefetchScalarGridSpec(
            num_scalar_prefetch=2,
            grid=(N_F, NT),
            in_specs=[
                pl.BlockSpec((P, D_MODEL), lambda f, i, te, tv: (0, 0)),
                pl.BlockSpec((1, F_BLK, D_MODEL),
                             lambda f, i, te, tv: (te[i], f, 0)),
                pl.BlockSpec((1, F_BLK, D_MODEL),
                             lambda f, i, te, tv: (te[i], f, 0)),
                pl.BlockSpec((1, D_MODEL, F_BLK),
                             lambda f, i, te, tv: (te[i], 0, f)),
            ],
            out_specs=pl.BlockSpec(
                (TT, D_MODEL),
                lambda f, i, te, tv: (jnp.where(f == N_F - 1, i, NT), 0)),
            scratch_shapes=[
                pltpu.VMEM((P, D_MODEL), jnp.bfloat16),
                pltpu.VMEM((F_BLK, D_MODEL), jnp.bfloat16),
                pltpu.VMEM((F_BLK, D_MODEL), jnp.bfloat16),
                pltpu.VMEM((D_MODEL, F_BLK), jnp.bfloat16),
            ],
        ),
        out_shape=jax.ShapeDtypeStruct((P + TT, D_MODEL), jnp.bfloat16),
        interpret=interpret,
    )(te, tv, xs, W1, W3, W2)

    out = pl.pallas_call(
        _combine_kernel,
        grid=(T // TC_CHUNK,),
        in_specs=[
            pl.BlockSpec((1, P), lambda t: (0, 0)),
            pl.BlockSpec((1, P), lambda t: (0, 0)),
            pl.BlockSpec((P, D_MODEL), lambda t: (0, 0)),
        ],
        out_specs=pl.BlockSpec((TC_CHUNK, D_MODEL), lambda t: (t, 0)),
        out_shape=jax.ShapeDtypeStruct((T, D_MODEL), jnp.float32),
        scratch_shapes=[pltpu.VMEM((TC_CHUNK, P), jnp.bfloat16)],
        interpret=interpret,
    )(tok2d, w2d, ys_padded)
    return out


def kernel(hidden_states, Wg, W1, W3, W2):
    b, s, d = hidden_states.shape
    x = hidden_states.reshape(-1, d)
    out = _run(x, Wg, W1, W3, W2)
    return out.reshape(b, s, d)
